# baseline (speedup 1.0000x reference)
import jax
import jax.numpy as jnp
from jax import lax
from jax.experimental import pallas as pl
from jax.experimental.pallas import tpu as pltpu
from jax.experimental.pallas import tpu_sc as plsc

BATCH = 1024
S = 7
N = 20
CELLS = BATCH * S * S
NC = 2
NS = 16
NW = NC * NS
CPT = CELLS // NW
GROUPS = CPT // 16
TCB = 128
GRID = BATCH // TCB
BLKC = TCB * S * S
WIN = 1664
Sf = 7.0


def _sq(x):
    return x * x


def _sqrt16(x):
    xi = plsc.bitcast(x, jnp.int32)
    yi = jnp.int32(0x5F3759DF) - lax.shift_right_arithmetic(xi, 1)
    y = plsc.bitcast(yi, jnp.float32)
    y = y * (1.5 - 0.5 * x * y * y)
    y = y * (1.5 - 0.5 * x * y * y)
    y = y * (1.5 - 0.5 * x * y * y)
    return jnp.where(x == 0.0, 0.0, x * y)


def _tc_body(p_ref, t_ref, po_ref, to_ref):
    xp = p_ref[...].reshape(BLKC, N)
    xt = t_ref[...].reshape(BLKC, N)
    po_ref[pl.ds(0, N), :] = jnp.transpose(xp)
    to_ref[pl.ds(0, N), :] = jnp.transpose(xt)


def _tc_transpose(pred_4d, targ_4d):
    return pl.pallas_call(
        _tc_body,
        grid=(GRID,),
        in_specs=[
            pl.BlockSpec((TCB, S, S, N), lambda i: (i, 0, 0, 0)),
            pl.BlockSpec((TCB, S, S, N), lambda i: (i, 0, 0, 0)),
        ],
        out_specs=[
            pl.BlockSpec((32, BLKC), lambda i: (0, i)),
            pl.BlockSpec((32, BLKC), lambda i: (0, i)),
        ],
        out_shape=[
            jax.ShapeDtypeStruct((32, CELLS), jnp.float32),
            jax.ShapeDtypeStruct((32, CELLS), jnp.float32),
        ],
    )(pred_4d, targ_4d)


def _body(pred_hbm, targ_hbm, out_hbm, pred_v, targ_v, acc_v):
    wid = lax.axis_index("s") * NC + lax.axis_index("c")
    col0 = wid * CPT
    base = lax.bitwise_and(col0, jnp.int32(~127))
    base = pl.multiple_of(base, 128)
    off = col0 - base

    def load_rows(c, carry):
        pltpu.sync_copy(pred_hbm.at[c, pl.ds(base, WIN)],
                        pred_v.at[c])
        pltpu.sync_copy(targ_hbm.at[c, pl.ds(base, WIN)],
                        targ_v.at[c])
        return carry

    lax.fori_loop(0, N, load_rows, 0)

    def group(g, acc):
        q = off + g * 16

        def pch(c):
            return pred_v[c, pl.ds(q, 16)]

        def tch(c):
            return targ_v[c, pl.ds(q, 16)]

        p = [pch(c) for c in range(10)]
        t = [tch(c) for c in range(10)]
        t4 = t[4]
        m = jnp.where(t4 > 0.0, 1.0, 0.0)
        l_noobj = jnp.where(t4 == 0.0,
                            _sq(p[4] - t4) + _sq(p[9] - t[9]),
                            0.0)
        l_class = _sq(pch(10) - tch(10))
        for c in range(11, 20):
            l_class = l_class + _sq(pch(c) - tch(c))
        C7 = jnp.float32(1.0 / Sf)
        tx = t[2] * C7
        at0 = 0.5 * t[2]
        at1 = 0.5 * t[3]
        lt_t0 = tx - at0
        lt_t1 = tx - at1
        rb_t0 = tx + at0
        rb_t1 = tx + at1
        area2 = t[2] * t[3]
        px = p[2] * C7
        py = p[7] * C7
        inters = []
        denoms = []
        for b in (0, 1):
            h0 = 0.5 * p[5 * b + 2]
            h1 = 0.5 * p[5 * b + 3]
            w = jnp.maximum(jnp.minimum(px + h0, rb_t0)
                            - jnp.maximum(px - h0, lt_t0), 0.0)
            h = jnp.maximum(jnp.minimum(py + h1, rb_t1)
                            - jnp.maximum(py - h1, lt_t1), 0.0)
            inter = w * h
            area1 = p[5 * b + 2] * p[5 * b + 3]
            inters.append(inter)
            denoms.append(area1 + area2 - inter)
        sel = inters[0] * denoms[1] >= inters[1] * denoms[0]
        max_iou = jnp.where(sel, inters[0], inters[1]) \
            / jnp.where(sel, denoms[0], denoms[1])
        pr = [jnp.where(sel, p[j], p[5 + j]) for j in range(5)]
        tr = [jnp.where(sel, t[j], t[5 + j]) for j in range(4)]
        l_xy = _sq(pr[0] - tr[0]) + _sq(pr[1] - tr[1])
        l_wh = pr[2] + tr[2] - 2.0 * _sqrt16(pr[2] * tr[2]) \
            + pr[3] + tr[3] - 2.0 * _sqrt16(pr[3] * tr[3])
        l_obj = _sq(pr[4] - max_iou)
        return acc + (m * (5.0 * (l_xy + l_wh) + l_obj)
                      + l_class * m + 0.5 * l_noobj)

    acc = lax.fori_loop(0, GROUPS, group, jnp.zeros((16,), jnp.float32),
                        unroll=2)
    acc_v[...] = acc
    pltpu.sync_copy(acc_v, out_hbm.at[wid])


@jax.jit
def _yolo_sc(pred_4d, targ_4d):
    pred_t, targ_t = _tc_transpose(pred_4d, targ_4d)
    mesh = plsc.VectorSubcoreMesh(
        core_axis_name="c", subcore_axis_name="s",
        num_cores=NC, num_subcores=NS)
    run = pl.kernel(
        _body,
        out_type=jax.ShapeDtypeStruct((NW, 16), jnp.float32),
        mesh=mesh,
        scratch_types=[
            pltpu.VMEM((N, WIN), jnp.float32),
            pltpu.VMEM((N, WIN), jnp.float32),
            pltpu.VMEM((16,), jnp.float32),
        ],
        compiler_params=pltpu.CompilerParams(needs_layout_passes=False),
    )
    partials = run(pred_t, targ_t)
    return jnp.sum(partials) * (1.0 / BATCH)


def kernel(pred_tensor, target_tensor):
    return _yolo_sc(pred_tensor, target_tensor)


# speedup vs baseline: 1.1643x; 1.1643x over previous
import jax
import jax.numpy as jnp
from jax import lax
from jax.experimental import pallas as pl
from jax.experimental.pallas import tpu as pltpu
from jax.experimental.pallas import tpu_sc as plsc

BATCH = 1024
S = 7
N = 20
CELLS = BATCH * S * S
NC = 2
NS = 16
NW = NC * NS
CPT = CELLS // NW
GROUPS = CPT // 16
TCB = 128
GRID = BATCH // TCB
BLKC = TCB * S * S
WIN = 1664
Sf = 7.0


def _sq(x):
    return x * x


def _sqrt16(x):
    xi = plsc.bitcast(x, jnp.int32)
    yi = jnp.int32(0x5F3759DF) - lax.shift_right_arithmetic(xi, 1)
    y = plsc.bitcast(yi, jnp.float32)
    y = y * (1.5 - 0.5 * x * y * y)
    y = y * (1.5 - 0.5 * x * y * y)
    y = y * (1.5 - 0.5 * x * y * y)
    return jnp.where(x == 0.0, 0.0, x * y)


def _tc_body(p_ref, t_ref, po_ref, to_ref):
    xp = p_ref[...].reshape(BLKC, N)
    xt = t_ref[...].reshape(BLKC, N)
    po_ref[pl.ds(0, N), :] = jnp.transpose(xp)
    to_ref[pl.ds(0, N), :] = jnp.transpose(xt)


def _tc_transpose(pred_4d, targ_4d):
    return pl.pallas_call(
        _tc_body,
        grid=(GRID,),
        in_specs=[
            pl.BlockSpec((TCB, S, S, N), lambda i: (i, 0, 0, 0)),
            pl.BlockSpec((TCB, S, S, N), lambda i: (i, 0, 0, 0)),
        ],
        out_specs=[
            pl.BlockSpec((32, BLKC), lambda i: (0, i)),
            pl.BlockSpec((32, BLKC), lambda i: (0, i)),
        ],
        out_shape=[
            jax.ShapeDtypeStruct((32, CELLS), jnp.float32),
            jax.ShapeDtypeStruct((32, CELLS), jnp.float32),
        ],
        compiler_params=pltpu.CompilerParams(
            allow_input_fusion=[True, True]),
    )(pred_4d, targ_4d)


def _body(pred_hbm, targ_hbm, out_hbm, pred_v, targ_v, acc_v):
    wid = lax.axis_index("s") * NC + lax.axis_index("c")
    col0 = wid * CPT
    base = lax.bitwise_and(col0, jnp.int32(~127))
    base = pl.multiple_of(base, 128)
    off = col0 - base

    for r in range(0, 24, 8):
        pltpu.sync_copy(pred_hbm.at[pl.ds(r, 8), pl.ds(base, WIN)],
                        pred_v.at[pl.ds(r, 8)])
        pltpu.sync_copy(targ_hbm.at[pl.ds(r, 8), pl.ds(base, WIN)],
                        targ_v.at[pl.ds(r, 8)])

    def group(g, acc):
        q = off + g * 16

        def pch(c):
            return pred_v[c, pl.ds(q, 16)]

        def tch(c):
            return targ_v[c, pl.ds(q, 16)]

        p = [pch(c) for c in range(10)]
        t = [tch(c) for c in range(10)]
        t4 = t[4]
        m = jnp.where(t4 > 0.0, 1.0, 0.0)
        l_noobj = jnp.where(t4 == 0.0,
                            _sq(p[4] - t4) + _sq(p[9] - t[9]),
                            0.0)
        l_class = _sq(pch(10) - tch(10))
        for c in range(11, 20):
            l_class = l_class + _sq(pch(c) - tch(c))
        C7 = jnp.float32(1.0 / Sf)
        tx = t[2] * C7
        at0 = 0.5 * t[2]
        at1 = 0.5 * t[3]
        lt_t0 = tx - at0
        lt_t1 = tx - at1
        rb_t0 = tx + at0
        rb_t1 = tx + at1
        area2 = t[2] * t[3]
        px = p[2] * C7
        py = p[7] * C7
        inters = []
        denoms = []
        for b in (0, 1):
            h0 = 0.5 * p[5 * b + 2]
            h1 = 0.5 * p[5 * b + 3]
            w = jnp.maximum(jnp.minimum(px + h0, rb_t0)
                            - jnp.maximum(px - h0, lt_t0), 0.0)
            h = jnp.maximum(jnp.minimum(py + h1, rb_t1)
                            - jnp.maximum(py - h1, lt_t1), 0.0)
            inter = w * h
            area1 = p[5 * b + 2] * p[5 * b + 3]
            inters.append(inter)
            denoms.append(area1 + area2 - inter)
        sel = inters[0] * denoms[1] >= inters[1] * denoms[0]
        max_iou = jnp.where(sel, inters[0], inters[1]) \
            / jnp.where(sel, denoms[0], denoms[1])
        pr = [jnp.where(sel, p[j], p[5 + j]) for j in range(5)]
        tr = [jnp.where(sel, t[j], t[5 + j]) for j in range(4)]
        l_xy = _sq(pr[0] - tr[0]) + _sq(pr[1] - tr[1])
        l_wh = pr[2] + tr[2] - 2.0 * _sqrt16(pr[2] * tr[2]) \
            + pr[3] + tr[3] - 2.0 * _sqrt16(pr[3] * tr[3])
        l_obj = _sq(pr[4] - max_iou)
        return acc + (m * (5.0 * (l_xy + l_wh) + l_obj)
                      + l_class * m + 0.5 * l_noobj)

    acc = lax.fori_loop(0, GROUPS, group, jnp.zeros((16,), jnp.float32),
                        unroll=2)
    acc_v[...] = acc
    pltpu.sync_copy(acc_v, out_hbm.at[wid])


@jax.jit
def _yolo_sc(pred_4d, targ_4d):
    pred_t, targ_t = _tc_transpose(pred_4d, targ_4d)
    mesh = plsc.VectorSubcoreMesh(
        core_axis_name="c", subcore_axis_name="s",
        num_cores=NC, num_subcores=NS)
    run = pl.kernel(
        _body,
        out_type=jax.ShapeDtypeStruct((NW, 16), jnp.float32),
        mesh=mesh,
        scratch_types=[
            pltpu.VMEM((24, WIN), jnp.float32),
            pltpu.VMEM((24, WIN), jnp.float32),
            pltpu.VMEM((16,), jnp.float32),
        ],
        compiler_params=pltpu.CompilerParams(needs_layout_passes=False),
    )
    partials = run(pred_t, targ_t)
    return jnp.sum(partials) * (1.0 / BATCH)


def kernel(pred_tensor, target_tensor):
    return _yolo_sc(pred_tensor, target_tensor)
